# Initial kernel scaffold; baseline (speedup 1.0000x reference)
import jax
import jax.numpy as jnp
from jax.experimental import pallas as pl


def kernel(inputs, table):
    raise NotImplementedError("write your pallas kernel here")


# speedup vs baseline: 1.7983x; 1.7983x over previous
import functools

import jax
import jax.numpy as jnp
from jax import lax
from jax.experimental import pallas as pl
from jax.experimental.pallas import tpu as pltpu
from jax.experimental.pallas import tpu_sc as plsc

_D = 64


@functools.cache
def _make_gather(B: int, V: int, D: int):
    info = plsc.get_sparse_core_info()
    nw = info.num_cores * info.num_subcores
    b_per_w = B // nw
    chunk = 512
    n_chunks = b_per_w // chunk
    mesh = plsc.VectorSubcoreMesh(core_axis_name="c", subcore_axis_name="s")

    @functools.partial(
        pl.kernel,
        mesh=mesh,
        out_type=jax.ShapeDtypeStruct((B, D), jnp.float32),
        scratch_types=[
            pltpu.VMEM((chunk,), jnp.int32),
            pltpu.VMEM((chunk, D), jnp.float32),
            pltpu.SemaphoreType.DMA,
        ],
        compiler_params=pltpu.CompilerParams(use_tc_tiling_on_sc=False),
    )
    def gather_kernel(table_hbm, idx_hbm, out_hbm, idx_v, rows_v, sem):
        wid = lax.axis_index("s") * info.num_cores + lax.axis_index("c")
        base = wid * b_per_w

        def body(g, carry):
            off = base + g * chunk
            pltpu.sync_copy(idx_hbm.at[pl.ds(off, chunk)], idx_v)
            pltpu.async_copy(table_hbm.at[idx_v], rows_v, sem).wait()
            pltpu.sync_copy(rows_v, out_hbm.at[pl.ds(off, chunk)])
            return carry

        lax.fori_loop(0, n_chunks, body, 0)

    return gather_kernel


def kernel(inputs, table):
    b, h = inputs.shape
    v, d = table.shape
    idx = inputs.reshape(-1).astype(jnp.int32)
    out = _make_gather(b * h, v, d)(table, idx)
    return out.reshape(b, h, d)


# speedup vs baseline: 1.8707x; 1.0402x over previous
import functools

import jax
import jax.numpy as jnp
from jax import lax
from jax.experimental import pallas as pl
from jax.experimental.pallas import tpu as pltpu
from jax.experimental.pallas import tpu_sc as plsc

_NBUF = 2
_CHUNK = 512


@functools.cache
def _make_gather(B: int, V: int, D: int):
    info = plsc.get_sparse_core_info()
    nw = info.num_cores * info.num_subcores
    b_per_w = B // nw
    chunk = _CHUNK
    nbuf = _NBUF
    n_steps = b_per_w // (chunk * nbuf)
    assert b_per_w % (chunk * nbuf) == 0
    mesh = plsc.VectorSubcoreMesh(core_axis_name="c", subcore_axis_name="s")

    @functools.partial(
        pl.kernel,
        mesh=mesh,
        out_type=jax.ShapeDtypeStruct((B, D), jnp.float32),
        scratch_types=[
            pltpu.VMEM((b_per_w,), jnp.int32),
            [pltpu.VMEM((chunk, D), jnp.float32) for _ in range(nbuf)],
            [pltpu.SemaphoreType.DMA for _ in range(nbuf)],
            [pltpu.SemaphoreType.DMA for _ in range(nbuf)],
        ],
        compiler_params=pltpu.CompilerParams(use_tc_tiling_on_sc=False),
    )
    def gather_kernel(table_hbm, idx_hbm, out_hbm, idx_v, rows, sem_g, sem_o):
        wid = lax.axis_index("s") * info.num_cores + lax.axis_index("c")
        base = wid * b_per_w
        pltpu.sync_copy(idx_hbm.at[pl.ds(base, b_per_w)], idx_v)

        def gather(g, b):
            pltpu.async_copy(
                table_hbm.at[idx_v.at[pl.ds(g * chunk, chunk)]], rows[b], sem_g[b]
            )

        def writeout(g, b):
            pltpu.async_copy(rows[b], out_hbm.at[pl.ds(base + g * chunk, chunk)],
                             sem_o[b])

        for b in range(nbuf):
            gather(b, b)
            pltpu.make_async_copy(
                table_hbm.at[idx_v.at[pl.ds(0, chunk)]], rows[b], sem_g[b]
            ).wait()
            writeout(b, b)

        def body(s, carry):
            for b in range(nbuf):
                g = s * nbuf + b
                pltpu.make_async_copy(
                    rows[b], out_hbm.at[pl.ds(base, chunk)], sem_o[b]
                ).wait()
                gather(g, b)
                pltpu.make_async_copy(
                    table_hbm.at[idx_v.at[pl.ds(0, chunk)]], rows[b], sem_g[b]
                ).wait()
                writeout(g, b)
            return carry

        lax.fori_loop(1, n_steps, body, 0)
        for b in range(nbuf):
            pltpu.make_async_copy(
                rows[b], out_hbm.at[pl.ds(base, chunk)], sem_o[b]
            ).wait()

    return gather_kernel


def kernel(inputs, table):
    b, h = inputs.shape
    v, d = table.shape
    idx = inputs.reshape(-1).astype(jnp.int32)
    out = _make_gather(b * h, v, d)(table, idx)
    return out.reshape(b, h, d)


# speedup vs baseline: 1.8791x; 1.0045x over previous
import functools

import jax
import jax.numpy as jnp
from jax import lax
from jax.experimental import pallas as pl
from jax.experimental.pallas import tpu as pltpu
from jax.experimental.pallas import tpu_sc as plsc

_ROWS_PER_BLK = 16
_NBUF = 2


@functools.cache
def _make_gather(B: int, H: int, V: int, D: int):
    info = plsc.get_sparse_core_info()
    nw = info.num_cores * info.num_subcores
    rows_w = B // nw
    blk = _ROWS_PER_BLK
    nbuf = _NBUF
    n_steps = rows_w // (blk * nbuf)
    assert rows_w % (blk * nbuf) == 0
    mesh = plsc.VectorSubcoreMesh(core_axis_name="c", subcore_axis_name="s")

    @functools.partial(
        pl.kernel,
        mesh=mesh,
        out_type=jax.ShapeDtypeStruct((B, H, D), jnp.float32),
        scratch_types=[
            pltpu.VMEM((rows_w, H), jnp.int32),
            [pltpu.VMEM((blk, H, D), jnp.float32) for _ in range(nbuf)],
            [pltpu.SemaphoreType.DMA for _ in range(nbuf)],
            [pltpu.SemaphoreType.DMA for _ in range(nbuf)],
        ],
        compiler_params=pltpu.CompilerParams(use_tc_tiling_on_sc=False),
    )
    def gather_kernel(table_hbm, idx_hbm, out_hbm, idx_v, rows, sem_g, sem_o):
        wid = lax.axis_index("s") * info.num_cores + lax.axis_index("c")
        base = wid * rows_w
        pltpu.sync_copy(idx_hbm.at[pl.ds(base, rows_w)], idx_v)

        def gather(g, b):
            for j in range(blk):
                pltpu.async_copy(
                    table_hbm.at[idx_v.at[g * blk + j]], rows[b].at[j], sem_g[b]
                )

        def wait_gather(b):
            for j in range(blk):
                pltpu.make_async_copy(
                    table_hbm.at[idx_v.at[0]], rows[b].at[j], sem_g[b]
                ).wait()

        def writeout(g, b):
            pltpu.async_copy(
                rows[b], out_hbm.at[pl.ds(base + g * blk, blk)], sem_o[b]
            )

        def wait_out(b):
            pltpu.make_async_copy(
                rows[b], out_hbm.at[pl.ds(base, blk)], sem_o[b]
            ).wait()

        for b in range(nbuf):
            gather(b, b)
            wait_gather(b)
            writeout(b, b)

        def body(s, carry):
            for b in range(nbuf):
                g = s * nbuf + b
                wait_out(b)
                gather(g, b)
                wait_gather(b)
                writeout(g, b)
            return carry

        lax.fori_loop(1, n_steps, body, 0)
        for b in range(nbuf):
            wait_out(b)

    return gather_kernel


def kernel(inputs, table):
    b, h = inputs.shape
    v, d = table.shape
    return _make_gather(b, h, v, d)(table, inputs.astype(jnp.int32))
